# baseline (speedup 1.0000x reference)
import jax
import jax.numpy as jnp
from jax.experimental import pallas as pl
from jax.experimental.pallas import tpu as pltpu

N_EXPERTS = 8
TOPK = 2
TILE = 512


def _router_body(x_ref, w_ref, vals_ref, idx_ref):
    x = x_ref[...]
    w = w_ref[...]
    s = jax.lax.dot_general(
        x, w,
        dimension_numbers=(((1,), (1,)), ((), ())),
        preferred_element_type=jnp.float32,
        precision=jax.lax.Precision.DEFAULT,
    )
    lanes = jax.lax.broadcasted_iota(jnp.int32, s.shape, 1)
    neg_big = jnp.float32(-3.0e38)
    m1 = jnp.max(s, axis=-1, keepdims=True)
    i1 = jnp.min(jnp.where(s == m1, lanes, N_EXPERTS), axis=-1, keepdims=True)
    s2 = jnp.where(lanes == i1, neg_big, s)
    m2 = jnp.max(s2, axis=-1, keepdims=True)
    i2 = jnp.min(jnp.where(s2 == m2, lanes, N_EXPERTS), axis=-1, keepdims=True)
    e2 = jnp.exp(m2 - m1)
    denom = 1.0 + e2
    v1 = 1.0 / denom
    v2 = e2 / denom
    vals_ref[...] = jnp.concatenate([v1, v2], axis=-1)
    idx_ref[...] = jnp.concatenate([i1, i2], axis=-1)


def kernel(x, W_router):
    batch, seqlen, hidden = x.shape
    n_tokens = batch * seqlen
    x_flat = x.reshape(n_tokens, hidden)
    grid = (n_tokens // TILE,)
    vals, idx = pl.pallas_call(
        _router_body,
        grid=grid,
        in_specs=[
            pl.BlockSpec((TILE, hidden), lambda i: (i, 0)),
            pl.BlockSpec((N_EXPERTS, hidden), lambda i: (0, 0)),
        ],
        out_specs=[
            pl.BlockSpec((TILE, TOPK), lambda i: (i, 0)),
            pl.BlockSpec((TILE, TOPK), lambda i: (i, 0)),
        ],
        out_shape=[
            jax.ShapeDtypeStruct((n_tokens, TOPK), jnp.float32),
            jax.ShapeDtypeStruct((n_tokens, TOPK), jnp.int32),
        ],
    )(x_flat, W_router)
    return (vals, idx)


# speedup vs baseline: 1.0161x; 1.0161x over previous
import jax
import jax.numpy as jnp
from jax.experimental import pallas as pl
from jax.experimental.pallas import tpu as pltpu

N_EXPERTS = 8
TOPK = 2
TILE = 1024


def _router_body(x_ref, w_ref, vals_ref, idx_ref):
    x = x_ref[...]
    w = w_ref[...]
    s = jax.lax.dot_general(
        x, w,
        dimension_numbers=(((1,), (1,)), ((), ())),
        preferred_element_type=jnp.float32,
        precision=jax.lax.Precision.DEFAULT,
    )
    lanes = jax.lax.broadcasted_iota(jnp.int32, s.shape, 1)
    neg_big = jnp.float32(-3.0e38)
    m1 = jnp.max(s, axis=-1, keepdims=True)
    i1 = jnp.min(jnp.where(s == m1, lanes, N_EXPERTS), axis=-1, keepdims=True)
    s2 = jnp.where(lanes == i1, neg_big, s)
    m2 = jnp.max(s2, axis=-1, keepdims=True)
    i2 = jnp.min(jnp.where(s2 == m2, lanes, N_EXPERTS), axis=-1, keepdims=True)
    e2 = jnp.exp(m2 - m1)
    denom = 1.0 + e2
    v1 = 1.0 / denom
    v2 = e2 / denom
    vals_ref[...] = jnp.concatenate([v1, v2], axis=-1)
    idx_ref[...] = jnp.concatenate([i1, i2], axis=-1)


def kernel(x, W_router):
    batch, seqlen, hidden = x.shape
    n_tokens = batch * seqlen
    x_flat = x.reshape(n_tokens, hidden)
    grid = (n_tokens // TILE,)
    vals, idx = pl.pallas_call(
        _router_body,
        grid=grid,
        in_specs=[
            pl.BlockSpec((TILE, hidden), lambda i: (i, 0)),
            pl.BlockSpec((N_EXPERTS, hidden), lambda i: (0, 0)),
        ],
        out_specs=[
            pl.BlockSpec((TILE, TOPK), lambda i: (i, 0)),
            pl.BlockSpec((TILE, TOPK), lambda i: (i, 0)),
        ],
        out_shape=[
            jax.ShapeDtypeStruct((n_tokens, TOPK), jnp.float32),
            jax.ShapeDtypeStruct((n_tokens, TOPK), jnp.int32),
        ],
    )(x_flat, W_router)
    return (vals, idx)
